# Initial kernel scaffold; baseline (speedup 1.0000x reference)
import jax
import jax.numpy as jnp
from jax.experimental import pallas as pl


def kernel(theta, dist, ins_feature, W1, b1, W2, b2, W3, b3, W4, b4, gamma, beta):
    raise NotImplementedError("write your pallas kernel here")


# speedup vs baseline: 4.3834x; 4.3834x over previous
import jax
import jax.numpy as jnp
from jax import lax
from jax.experimental import pallas as pl
from jax.experimental.pallas import tpu as pltpu

_B, _N, _K = 32, 32, 8192
_EMB = 128
_LOCAL = 32
_PEN = -100000.0
_R = _B * _N
_RB = 32


def _topk_body(dist_ref, theta_ref, sd_ref, th_ref, idx_ref, d_scr):
    d_scr[...] = dist_ref[...]
    kio = lax.broadcasted_iota(jnp.int32, (_RB, _K), 1)
    cio = lax.broadcasted_iota(jnp.int32, (_RB, _LOCAL), 1)
    th_blk = theta_ref[...]

    def body(j, carry):
        sd, th, ix = carry
        d = d_scr[...]
        mn = jnp.min(d, axis=1, keepdims=True)
        pos = jnp.min(jnp.where(d == mn, kio, _K), axis=1, keepdims=True)
        picked = kio == pos
        tj = jnp.sum(jnp.where(picked, th_blk, 0.0), axis=1, keepdims=True)
        d_scr[...] = jnp.where(picked, jnp.inf, d)
        sel = cio == j
        return (jnp.where(sel, mn, sd), jnp.where(sel, tj, th),
                jnp.where(sel, pos, ix))

    z = jnp.zeros((_RB, _LOCAL), jnp.float32)
    sd, th, ix = lax.fori_loop(
        0, _LOCAL, body, (z, z, jnp.zeros((_RB, _LOCAL), jnp.int32)))
    sd_ref[...] = sd / sd[:, _LOCAL - 1:]
    th_ref[...] = th
    idx_ref[...] = ix


def _mlp_body(sd_ref, th_ref, i0_ref, i1_ref, w1a_ref, w1b_ref, w1cd_ref,
              b1_ref, w2_ref, b2_ref, w3_ref, b3_ref, w4_ref, b4_ref,
              g_ref, be_ref, out_ref):
    dn = (((1,), (1,)), ((), ()))
    f32 = jnp.float32
    sd = sd_ref[...]
    emb = lax.dot_general(sd, w1a_ref[...], dn, preferred_element_type=f32)
    emb = emb + lax.dot_general(th_ref[...], w1b_ref[...], dn,
                                preferred_element_type=f32)
    emb = emb + i0_ref[...] * w1cd_ref[0:1, :] + i1_ref[...] * w1cd_ref[1:2, :]
    emb = jnp.maximum(emb + b1_ref[...], 0.0)
    h = jnp.maximum(
        lax.dot_general(emb, w2_ref[...], dn, preferred_element_type=f32)
        + b2_ref[...], 0.0)
    seg = (lax.broadcasted_iota(jnp.int32, (_B, _R), 1) // _N
           == lax.broadcasted_iota(jnp.int32, (_B, _R), 0))
    sm = jnp.where(seg, 1.0 / _N, 0.0)
    tt = jnp.where(seg, 1.0, 0.0)
    mean = lax.dot_general(sm, h, (((1,), (0,)), ((), ())),
                           preferred_element_type=f32)
    meanf = lax.dot_general(tt, mean, (((0,), (0,)), ((), ())),
                            preferred_element_type=f32)
    c = h - meanf
    var = lax.dot_general(sm, c * c, (((1,), (0,)), ((), ())),
                          preferred_element_type=f32)
    varf = lax.dot_general(tt, var, (((0,), (0,)), ((), ())),
                           preferred_element_type=f32)
    h = c * lax.rsqrt(varf + 1e-5) * g_ref[...] + be_ref[...]
    emb2 = jnp.maximum(
        lax.dot_general(h, w3_ref[...], dn, preferred_element_type=f32)
        + b3_ref[...], 0.0)
    out = lax.dot_general(emb2, w4_ref[...], dn, preferred_element_type=f32)
    out_ref[...] = out + b4_ref[...] - sd


def _scatter_body(v_ref, idx_ref, out_ref):
    kio = lax.broadcasted_iota(jnp.int32, (_RB, _K), 1)
    cio = lax.broadcasted_iota(jnp.int32, (_RB, _LOCAL), 1)
    v = v_ref[...]
    ix = idx_ref[...]

    def body(j, acc):
        sel = cio == j
        ij = jnp.sum(jnp.where(sel, ix, 0), axis=1, keepdims=True)
        vj = jnp.sum(jnp.where(sel, v, 0.0), axis=1, keepdims=True)
        return jnp.where(kio == ij, vj, acc)

    out_ref[...] = lax.fori_loop(
        0, _LOCAL, body, jnp.full((_RB, _K), _PEN, jnp.float32))


def kernel(theta, dist, ins_feature, W1, b1, W2, b2, W3, b3, W4, b4,
           gamma, beta):
    d2 = dist.reshape(_R, _K)
    t2 = theta.reshape(_R, _K)
    nblk = _R // _RB

    sd, th, ix = pl.pallas_call(
        _topk_body,
        grid=(nblk,),
        in_specs=[pl.BlockSpec((_RB, _K), lambda i: (i, 0)),
                  pl.BlockSpec((_RB, _K), lambda i: (i, 0))],
        out_specs=[pl.BlockSpec((_RB, _LOCAL), lambda i: (i, 0))] * 3,
        out_shape=[jax.ShapeDtypeStruct((_R, _LOCAL), jnp.float32),
                   jax.ShapeDtypeStruct((_R, _LOCAL), jnp.float32),
                   jax.ShapeDtypeStruct((_R, _LOCAL), jnp.int32)],
        scratch_shapes=[pltpu.VMEM((_RB, _K), jnp.float32)],
    )(d2, t2)

    i0 = ins_feature[0].reshape(_R, 1)
    i1 = ins_feature[1].reshape(_R, 1)
    w1a = W1[:, :_LOCAL]
    w1b = W1[:, _LOCAL:2 * _LOCAL]
    w1cd = W1[:, 2 * _LOCAL:].T

    vals = pl.pallas_call(
        _mlp_body,
        out_shape=jax.ShapeDtypeStruct((_R, _LOCAL), jnp.float32),
    )(sd, th, i0, i1, w1a, w1b, w1cd,
      b1.reshape(1, _EMB), W2, b2.reshape(1, 2 * _EMB), W3,
      b3.reshape(1, _EMB), W4, b4.reshape(1, _LOCAL),
      gamma.reshape(1, 2 * _EMB), beta.reshape(1, 2 * _EMB))

    out = pl.pallas_call(
        _scatter_body,
        grid=(nblk,),
        in_specs=[pl.BlockSpec((_RB, _LOCAL), lambda i: (i, 0)),
                  pl.BlockSpec((_RB, _LOCAL), lambda i: (i, 0))],
        out_specs=pl.BlockSpec((_RB, _K), lambda i: (i, 0)),
        out_shape=jax.ShapeDtypeStruct((_R, _K), jnp.float32),
    )(vals, ix)
    return out.reshape(_B, _N, _K)


# speedup vs baseline: 6.4865x; 1.4798x over previous
import jax
import jax.numpy as jnp
from jax import lax
from jax.experimental import pallas as pl
from jax.experimental.pallas import tpu as pltpu

_B, _N, _K = 32, 32, 8192
_EMB = 128
_LOCAL = 32
_PEN = -100000.0
_R = _B * _N
_RB = 256


def _topk_body(dist_ref, theta_ref, sd_ref, th_ref, idx_ref, d_scr):
    d_scr[...] = dist_ref[...]
    kio = lax.broadcasted_iota(jnp.int32, (_RB, _K), 1)
    cio = lax.broadcasted_iota(jnp.int32, (_RB, _LOCAL), 1)
    th_blk = theta_ref[...]

    def body(j, carry):
        sd, th, ix = carry
        d = d_scr[...]
        mn = jnp.min(d, axis=1, keepdims=True)
        pos = jnp.min(jnp.where(d == mn, kio, _K), axis=1, keepdims=True)
        picked = kio == pos
        tj = jnp.sum(jnp.where(picked, th_blk, 0.0), axis=1, keepdims=True)
        d_scr[...] = jnp.where(picked, jnp.inf, d)
        sel = cio == j
        return (jnp.where(sel, mn, sd), jnp.where(sel, tj, th),
                jnp.where(sel, pos, ix))

    z = jnp.zeros((_RB, _LOCAL), jnp.float32)
    sd, th, ix = lax.fori_loop(
        0, _LOCAL, body, (z, z, jnp.zeros((_RB, _LOCAL), jnp.int32)))
    sd_ref[...] = sd / sd[:, _LOCAL - 1:]
    th_ref[...] = th
    idx_ref[...] = ix


def _mlp_body(sd_ref, th_ref, i0_ref, i1_ref, w1a_ref, w1b_ref, w1cd_ref,
              b1_ref, w2_ref, b2_ref, w3_ref, b3_ref, w4_ref, b4_ref,
              g_ref, be_ref, out_ref):
    dn = (((1,), (1,)), ((), ()))
    f32 = jnp.float32
    sd = sd_ref[...]
    emb = lax.dot_general(sd, w1a_ref[...], dn, preferred_element_type=f32)
    emb = emb + lax.dot_general(th_ref[...], w1b_ref[...], dn,
                                preferred_element_type=f32)
    emb = emb + i0_ref[...] * w1cd_ref[0:1, :] + i1_ref[...] * w1cd_ref[1:2, :]
    emb = jnp.maximum(emb + b1_ref[...], 0.0)
    h = jnp.maximum(
        lax.dot_general(emb, w2_ref[...], dn, preferred_element_type=f32)
        + b2_ref[...], 0.0)
    seg = (lax.broadcasted_iota(jnp.int32, (_B, _R), 1) // _N
           == lax.broadcasted_iota(jnp.int32, (_B, _R), 0))
    sm = jnp.where(seg, 1.0 / _N, 0.0)
    tt = jnp.where(seg, 1.0, 0.0)
    mean = lax.dot_general(sm, h, (((1,), (0,)), ((), ())),
                           preferred_element_type=f32)
    meanf = lax.dot_general(tt, mean, (((0,), (0,)), ((), ())),
                            preferred_element_type=f32)
    c = h - meanf
    var = lax.dot_general(sm, c * c, (((1,), (0,)), ((), ())),
                          preferred_element_type=f32)
    varf = lax.dot_general(tt, var, (((0,), (0,)), ((), ())),
                           preferred_element_type=f32)
    h = c * lax.rsqrt(varf + 1e-5) * g_ref[...] + be_ref[...]
    emb2 = jnp.maximum(
        lax.dot_general(h, w3_ref[...], dn, preferred_element_type=f32)
        + b3_ref[...], 0.0)
    out = lax.dot_general(emb2, w4_ref[...], dn, preferred_element_type=f32)
    out_ref[...] = out + b4_ref[...] - sd


def _scatter_body(v_ref, idx_ref, out_ref):
    kio = lax.broadcasted_iota(jnp.int32, (_RB, _K), 1)
    cio = lax.broadcasted_iota(jnp.int32, (_RB, _LOCAL), 1)
    v = v_ref[...]
    ix = idx_ref[...]

    def body(j, acc):
        sel = cio == j
        ij = jnp.sum(jnp.where(sel, ix, 0), axis=1, keepdims=True)
        vj = jnp.sum(jnp.where(sel, v, 0.0), axis=1, keepdims=True)
        return jnp.where(kio == ij, vj, acc)

    out_ref[...] = lax.fori_loop(
        0, _LOCAL, body, jnp.full((_RB, _K), _PEN, jnp.float32))


def kernel(theta, dist, ins_feature, W1, b1, W2, b2, W3, b3, W4, b4,
           gamma, beta):
    d2 = dist.reshape(_R, _K)
    t2 = theta.reshape(_R, _K)
    nblk = _R // _RB

    sd, th, ix = pl.pallas_call(
        _topk_body,
        grid=(nblk,),
        in_specs=[pl.BlockSpec((_RB, _K), lambda i: (i, 0)),
                  pl.BlockSpec((_RB, _K), lambda i: (i, 0))],
        out_specs=[pl.BlockSpec((_RB, _LOCAL), lambda i: (i, 0))] * 3,
        out_shape=[jax.ShapeDtypeStruct((_R, _LOCAL), jnp.float32),
                   jax.ShapeDtypeStruct((_R, _LOCAL), jnp.float32),
                   jax.ShapeDtypeStruct((_R, _LOCAL), jnp.int32)],
        scratch_shapes=[pltpu.VMEM((_RB, _K), jnp.float32)],
    )(d2, t2)

    i0 = ins_feature[0].reshape(_R, 1)
    i1 = ins_feature[1].reshape(_R, 1)
    w1a = W1[:, :_LOCAL]
    w1b = W1[:, _LOCAL:2 * _LOCAL]
    w1cd = W1[:, 2 * _LOCAL:].T

    vals = pl.pallas_call(
        _mlp_body,
        out_shape=jax.ShapeDtypeStruct((_R, _LOCAL), jnp.float32),
    )(sd, th, i0, i1, w1a, w1b, w1cd,
      b1.reshape(1, _EMB), W2, b2.reshape(1, 2 * _EMB), W3,
      b3.reshape(1, _EMB), W4, b4.reshape(1, _LOCAL),
      gamma.reshape(1, 2 * _EMB), beta.reshape(1, 2 * _EMB))

    out = pl.pallas_call(
        _scatter_body,
        grid=(nblk,),
        in_specs=[pl.BlockSpec((_RB, _LOCAL), lambda i: (i, 0)),
                  pl.BlockSpec((_RB, _LOCAL), lambda i: (i, 0))],
        out_specs=pl.BlockSpec((_RB, _K), lambda i: (i, 0)),
        out_shape=jax.ShapeDtypeStruct((_R, _K), jnp.float32),
    )(vals, ix)
    return out.reshape(_B, _N, _K)
